# Initial kernel scaffold; baseline (speedup 1.0000x reference)
import functools

import jax
import jax.numpy as jnp
from jax import lax
from jax.experimental import pallas as pl
from jax.experimental.pallas import tpu as pltpu
from jax.experimental.pallas import tpu_sc as plsc

_TN = 512


def _topk_body(x_ref, y_ref, i0_ref, i1_ref, i2_ref, *, n_y, n_taps):
    b = pl.program_id(0)
    xb = x_ref[0]
    yb = y_ref[0]
    dot = lax.dot_general(xb, yb, (((0,), (0,)), ((), ())),
                          preferred_element_type=jnp.float32)
    nx = jnp.sum(xb * xb, axis=0)[:, None]
    ny = jnp.sum(yb * yb, axis=0)[None, :]
    dist = jnp.sqrt(nx + ny - 2.0 * dot)
    iota = lax.broadcasted_iota(jnp.float32, dist.shape, 1)
    out_refs = (i0_ref, i1_ref, i2_ref)
    for k in range(n_taps):
        mn = jnp.min(dist, axis=1, keepdims=True)
        sel = jnp.where(dist == mn, iota, jnp.float32(n_y))
        idx_f = jnp.min(sel, axis=1, keepdims=True)
        out_refs[k][0, 0] = idx_f.astype(jnp.int32) + (b * n_taps + k) * n_y
        if k < n_taps - 1:
            dist = jnp.where(iota == idx_f, jnp.float32(jnp.inf), dist)


def _u_body(x_ref, wt_ref, b_ref, u_ref):
    k = pl.program_id(1)
    xb = x_ref[0]
    wk = wt_ref[0]
    u = lax.dot_general(xb, wk, (((0,), (0,)), ((), ())),
                        preferred_element_type=jnp.float32)
    u = u + jnp.where(k == 0, 1.0, 0.0).astype(jnp.float32) * b_ref[...]
    u_ref[0, 0] = u


def _make_sc_gather_sum(n_rows, d, n_workers, chunk):
    rpw = n_rows // n_workers
    n_ch = rpw // chunk
    mesh = plsc.VectorSubcoreMesh(core_axis_name="c", subcore_axis_name="s")

    @functools.partial(
        pl.kernel, mesh=mesh,
        out_type=jax.ShapeDtypeStruct((n_rows, d), jnp.float32),
        scratch_types=[
            pltpu.VMEM((chunk,), jnp.int32),
            pltpu.VMEM((chunk,), jnp.int32),
            pltpu.VMEM((chunk,), jnp.int32),
            pltpu.VMEM((chunk, d), jnp.float32),
            pltpu.VMEM((chunk, d), jnp.float32),
            pltpu.VMEM((chunk, d), jnp.float32),
            pltpu.VMEM((chunk, d), jnp.float32),
            pltpu.SemaphoreType.DMA,
        ],
    )
    def sc_kernel(tab_hbm, idx0_hbm, idx1_hbm, idx2_hbm, out_hbm,
                  i0_v, i1_v, i2_v, r0_v, r1_v, r2_v, out_v, sem):
        wid = lax.axis_index("s") * 2 + lax.axis_index("c")
        base = wid * rpw
        for ch in range(n_ch):
            row0 = base + ch * chunk
            pltpu.sync_copy(idx0_hbm.at[pl.ds(row0, chunk)], i0_v)
            pltpu.sync_copy(idx1_hbm.at[pl.ds(row0, chunk)], i1_v)
            pltpu.sync_copy(idx2_hbm.at[pl.ds(row0, chunk)], i2_v)
            c0 = pltpu.async_copy(tab_hbm.at[i0_v], r0_v, sem)
            c1 = pltpu.async_copy(tab_hbm.at[i1_v], r1_v, sem)
            c2 = pltpu.async_copy(tab_hbm.at[i2_v], r2_v, sem)
            c0.wait()
            c1.wait()
            c2.wait()

            def body(r, _):
                for g in range(d // 16):
                    s = pl.ds(g * 16, 16)
                    v = r0_v[r, s] + r1_v[r, s] + r2_v[r, s]
                    out_v[r, s] = jnp.maximum(v, jnp.float32(0.0))
                return 0

            lax.fori_loop(0, chunk, body, 0)
            pltpu.sync_copy(out_v, out_hbm.at[pl.ds(row0, chunk)])

    return sc_kernel


def kernel(x, y, W, b):
    B, C, N = x.shape
    M = y.shape[2]
    CO, _, K = W.shape
    NB = N // _TN

    i0, i1, i2 = pl.pallas_call(
        functools.partial(_topk_body, n_y=M, n_taps=K),
        grid=(B, NB),
        in_specs=[
            pl.BlockSpec((1, C, _TN), lambda bb, nb: (bb, 0, nb)),
            pl.BlockSpec((1, C, M), lambda bb, nb: (bb, 0, 0)),
        ],
        out_specs=[
            pl.BlockSpec((1, 1, _TN, 1), lambda bb, nb: (bb, nb, 0, 0)),
            pl.BlockSpec((1, 1, _TN, 1), lambda bb, nb: (bb, nb, 0, 0)),
            pl.BlockSpec((1, 1, _TN, 1), lambda bb, nb: (bb, nb, 0, 0)),
        ],
        out_shape=[jax.ShapeDtypeStruct((B, NB, _TN, 1), jnp.int32)] * 3,
    )(x, y)

    Wt = jnp.transpose(W, (2, 1, 0))
    b2 = b.reshape(1, CO)
    U = pl.pallas_call(
        _u_body,
        grid=(B, K),
        in_specs=[
            pl.BlockSpec((1, C, M), lambda bb, kk: (bb, 0, 0)),
            pl.BlockSpec((1, C, CO), lambda bb, kk: (kk, 0, 0)),
            pl.BlockSpec((1, CO), lambda bb, kk: (0, 0)),
        ],
        out_specs=pl.BlockSpec((1, 1, M, CO), lambda bb, kk: (bb, kk, 0, 0)),
        out_shape=jax.ShapeDtypeStruct((B, K, M, CO), jnp.float32),
    )(x, Wt, b2)

    table = U.reshape(B * K * M, CO)
    sc = _make_sc_gather_sum(B * N, CO, n_workers=32, chunk=128)
    out_rows = sc(table, i0.reshape(B * N), i1.reshape(B * N),
                  i2.reshape(B * N))
    return out_rows.reshape(B, N, CO).transpose(0, 2, 1)


# speedup vs baseline: 22.1886x; 22.1886x over previous
import functools

import jax
import jax.numpy as jnp
from jax import lax
from jax.experimental import pallas as pl
from jax.experimental.pallas import tpu as pltpu
from jax.experimental.pallas import tpu_sc as plsc

_TN = 512


def _topk_body(x_ref, y_ref, i0_ref, i1_ref, i2_ref, *, n_y, n_taps):
    b = pl.program_id(0)
    xb = x_ref[0]
    yb = y_ref[0]
    dot = lax.dot_general(xb, yb, (((0,), (0,)), ((), ())),
                          preferred_element_type=jnp.float32)
    nx = jnp.sum(xb * xb, axis=0)[:, None]
    ny = jnp.sum(yb * yb, axis=0)[None, :]
    dist = jnp.sqrt(nx + ny - 2.0 * dot)
    iota = lax.broadcasted_iota(jnp.int32, dist.shape, 1)
    out_refs = (i0_ref, i1_ref, i2_ref)
    for k in range(n_taps):
        mn = jnp.min(dist, axis=1, keepdims=True)
        sel = jnp.where(dist == mn, iota, jnp.int32(n_y))
        idx = jnp.min(sel, axis=1, keepdims=True)
        out_refs[k][0, 0] = idx + (b * n_taps + k) * n_y
        if k < n_taps - 1:
            dist = jnp.where(iota == idx, jnp.float32(jnp.inf), dist)


def _u_body(x_ref, wt_ref, b_ref, u_ref):
    k = pl.program_id(1)
    xb = x_ref[0]
    wk = wt_ref[0]
    u = lax.dot_general(xb, wk, (((0,), (0,)), ((), ())),
                        preferred_element_type=jnp.float32)
    u = u + jnp.where(k == 0, 1.0, 0.0).astype(jnp.float32) * b_ref[...]
    u_ref[0, 0] = u


def _make_sc_gather_sum(n_rows, d, n_workers, chunk):
    rpw = n_rows // n_workers
    n_ch = rpw // chunk
    mesh = plsc.VectorSubcoreMesh(core_axis_name="c", subcore_axis_name="s")

    @functools.partial(
        pl.kernel, mesh=mesh,
        out_type=jax.ShapeDtypeStruct((n_rows, d), jnp.float32),
        scratch_types=[
            pltpu.VMEM((chunk,), jnp.int32),
            pltpu.VMEM((chunk,), jnp.int32),
            pltpu.VMEM((chunk,), jnp.int32),
            pltpu.VMEM((chunk, d), jnp.float32),
            pltpu.VMEM((chunk, d), jnp.float32),
            pltpu.VMEM((chunk, d), jnp.float32),
            pltpu.VMEM((chunk, d), jnp.float32),
            pltpu.SemaphoreType.DMA,
        ],
    )
    def sc_kernel(tab_hbm, idx0_hbm, idx1_hbm, idx2_hbm, out_hbm,
                  i0_v, i1_v, i2_v, r0_v, r1_v, r2_v, out_v, sem):
        wid = lax.axis_index("s") * 2 + lax.axis_index("c")
        base = wid * rpw
        for ch in range(n_ch):
            row0 = base + ch * chunk
            pltpu.sync_copy(idx0_hbm.at[pl.ds(row0, chunk)], i0_v)
            pltpu.sync_copy(idx1_hbm.at[pl.ds(row0, chunk)], i1_v)
            pltpu.sync_copy(idx2_hbm.at[pl.ds(row0, chunk)], i2_v)
            c0 = pltpu.async_copy(tab_hbm.at[i0_v], r0_v, sem)
            c1 = pltpu.async_copy(tab_hbm.at[i1_v], r1_v, sem)
            c2 = pltpu.async_copy(tab_hbm.at[i2_v], r2_v, sem)
            c0.wait()
            c1.wait()
            c2.wait()

            def body(r, _):
                for g in range(d // 16):
                    s = pl.ds(g * 16, 16)
                    v = r0_v[r, s] + r1_v[r, s] + r2_v[r, s]
                    out_v[r, s] = jnp.maximum(v, jnp.float32(0.0))
                return 0

            lax.fori_loop(0, chunk, body, 0)
            pltpu.sync_copy(out_v, out_hbm.at[pl.ds(row0, chunk)])

    return sc_kernel


def kernel(x, y, W, b):
    B, C, N = x.shape
    M = y.shape[2]
    CO, _, K = W.shape
    NB = N // _TN

    i0, i1, i2 = pl.pallas_call(
        functools.partial(_topk_body, n_y=M, n_taps=K),
        grid=(B, NB),
        in_specs=[
            pl.BlockSpec((1, C, _TN), lambda bb, nb: (bb, 0, nb)),
            pl.BlockSpec((1, C, M), lambda bb, nb: (bb, 0, 0)),
        ],
        out_specs=[
            pl.BlockSpec((1, 1, _TN, 1), lambda bb, nb: (bb, nb, 0, 0)),
            pl.BlockSpec((1, 1, _TN, 1), lambda bb, nb: (bb, nb, 0, 0)),
            pl.BlockSpec((1, 1, _TN, 1), lambda bb, nb: (bb, nb, 0, 0)),
        ],
        out_shape=[jax.ShapeDtypeStruct((B, NB, _TN, 1), jnp.int32)] * 3,
    )(x, y)

    Wt = jnp.transpose(W, (2, 1, 0))
    b2 = b.reshape(1, CO)
    U = pl.pallas_call(
        _u_body,
        grid=(B, K),
        in_specs=[
            pl.BlockSpec((1, C, M), lambda bb, kk: (bb, 0, 0)),
            pl.BlockSpec((1, C, CO), lambda bb, kk: (kk, 0, 0)),
            pl.BlockSpec((1, CO), lambda bb, kk: (0, 0)),
        ],
        out_specs=pl.BlockSpec((1, 1, M, CO), lambda bb, kk: (bb, kk, 0, 0)),
        out_shape=jax.ShapeDtypeStruct((B, K, M, CO), jnp.float32),
    )(x, Wt, b2)

    table = U.reshape(B * K * M, CO)
    sc = _make_sc_gather_sum(B * N, CO, n_workers=32, chunk=128)
    out_rows = sc(table, i0.reshape(B * N), i1.reshape(B * N),
                  i2.reshape(B * N))
    return out_rows.reshape(B, N, CO).transpose(0, 2, 1)


# speedup vs baseline: 24.2048x; 1.0909x over previous
import functools

import jax
import jax.numpy as jnp
from jax import lax
from jax.experimental import pallas as pl
from jax.experimental.pallas import tpu as pltpu
from jax.experimental.pallas import tpu_sc as plsc

_TN = 512


def _topk_body(x_ref, y_ref, i0_ref, i1_ref, i2_ref, *, n_y, n_taps):
    b = pl.program_id(0)
    xb = x_ref[0]
    yb = y_ref[0]
    yb2 = -2.0 * yb
    dot2 = lax.dot_general(xb, yb2, (((0,), (0,)), ((), ())),
                           preferred_element_type=jnp.float32)
    nx = jnp.sum(xb * xb, axis=0)[:, None]
    ny = 0.25 * jnp.sum(yb2 * yb2, axis=0)[None, :]
    dist = jnp.sqrt((nx + ny) + dot2)
    iota = lax.broadcasted_iota(jnp.int32, dist.shape, 1)
    out_refs = (i0_ref, i1_ref, i2_ref)
    for k in range(n_taps):
        mn = jnp.min(dist, axis=1, keepdims=True)
        sel = jnp.where(dist == mn, iota, jnp.int32(n_y))
        idx = jnp.min(sel, axis=1, keepdims=True)
        out_refs[k][0, 0] = idx + (b * n_taps + k) * n_y
        if k < n_taps - 1:
            dist = jnp.where(iota == idx, jnp.float32(jnp.inf), dist)


def _u_body(x_ref, wt_ref, b_ref, u_ref):
    k = pl.program_id(1)
    xb = x_ref[0]
    wk = wt_ref[0]
    u = lax.dot_general(xb, wk, (((0,), (0,)), ((), ())),
                        preferred_element_type=jnp.float32)
    u = u + jnp.where(k == 0, 1.0, 0.0).astype(jnp.float32) * b_ref[...]
    u_ref[0, 0] = u


def _make_sc_gather_sum(n_rows, d, n_workers, chunk, n_taps):
    rpw = n_rows // n_workers
    n_ch = rpw // chunk
    mesh = plsc.VectorSubcoreMesh(core_axis_name="c", subcore_axis_name="s")
    rbuf = [pltpu.VMEM((chunk, d), jnp.float32)] * (2 * n_taps)
    obuf = [pltpu.VMEM((chunk, d), jnp.float32)] * 2

    @functools.partial(
        pl.kernel, mesh=mesh,
        out_type=jax.ShapeDtypeStruct((n_rows, d), jnp.float32),
        scratch_types=[pltpu.VMEM((n_taps, n_ch, chunk), jnp.int32)]
        + rbuf + obuf
        + [pltpu.SemaphoreType.DMA] * 2 + [pltpu.SemaphoreType.DMA] * 2,
    )
    def sc_kernel(tab_hbm, idx_hbm, out_hbm, idx_v, *bufs_and_sems):
        rbufs = [bufs_and_sems[b * n_taps:(b + 1) * n_taps] for b in range(2)]
        obufs = bufs_and_sems[2 * n_taps:2 * n_taps + 2]
        gsems = bufs_and_sems[2 * n_taps + 2:2 * n_taps + 4]
        ssems = bufs_and_sems[2 * n_taps + 4:2 * n_taps + 6]
        wid = lax.axis_index("s") * 2 + lax.axis_index("c")
        base = wid * rpw
        pltpu.sync_copy(idx_hbm.at[wid], idx_v)

        def fire(ch, b):
            return [pltpu.async_copy(tab_hbm.at[idx_v.at[k, ch]],
                                     rbufs[b][k], gsems[b])
                    for k in range(n_taps)]

        pending = {0: fire(0, 0)}
        stores = {}
        for ch in range(n_ch):
            b = ch % 2
            if ch + 1 < n_ch:
                pending[ch + 1] = fire(ch + 1, 1 - b)
            for c in pending.pop(ch):
                c.wait()
            if ch >= 2:
                stores.pop(ch - 2).wait()
            r0, r1, r2 = rbufs[b]
            out_v = obufs[b]

            def body(r, _):
                for g in range(d // 16):
                    s = pl.ds(g * 16, 16)
                    v = r0[r, s] + r1[r, s] + r2[r, s]
                    out_v[r, s] = jnp.maximum(v, jnp.float32(0.0))
                return 0

            lax.fori_loop(0, chunk, body, 0)
            stores[ch] = pltpu.async_copy(
                out_v, out_hbm.at[pl.ds(base + ch * chunk, chunk)], ssems[b])
        for c in stores.values():
            c.wait()

    return sc_kernel


def kernel(x, y, W, b):
    B, C, N = x.shape
    M = y.shape[2]
    CO, _, K = W.shape
    NB = N // _TN

    i0, i1, i2 = pl.pallas_call(
        functools.partial(_topk_body, n_y=M, n_taps=K),
        grid=(B, NB),
        in_specs=[
            pl.BlockSpec((1, C, _TN), lambda bb, nb: (bb, 0, nb)),
            pl.BlockSpec((1, C, M), lambda bb, nb: (bb, 0, 0)),
        ],
        out_specs=[
            pl.BlockSpec((1, 1, _TN, 1), lambda bb, nb: (bb, nb, 0, 0)),
            pl.BlockSpec((1, 1, _TN, 1), lambda bb, nb: (bb, nb, 0, 0)),
            pl.BlockSpec((1, 1, _TN, 1), lambda bb, nb: (bb, nb, 0, 0)),
        ],
        out_shape=[jax.ShapeDtypeStruct((B, NB, _TN, 1), jnp.int32)] * 3,
    )(x, y)

    Wt = jnp.transpose(W, (2, 1, 0))
    b2 = b.reshape(1, CO)
    U = pl.pallas_call(
        _u_body,
        grid=(B, K),
        in_specs=[
            pl.BlockSpec((1, C, M), lambda bb, kk: (bb, 0, 0)),
            pl.BlockSpec((1, C, CO), lambda bb, kk: (kk, 0, 0)),
            pl.BlockSpec((1, CO), lambda bb, kk: (0, 0)),
        ],
        out_specs=pl.BlockSpec((1, 1, M, CO), lambda bb, kk: (bb, kk, 0, 0)),
        out_shape=jax.ShapeDtypeStruct((B, K, M, CO), jnp.float32),
    )(x, Wt, b2)

    NWORK, CHUNK = 32, 64
    table = U.reshape(B * K * M, CO)
    rpw = B * N // NWORK
    idx_planes = jnp.stack(
        [i0.reshape(B * N), i1.reshape(B * N), i2.reshape(B * N)], axis=0
    ).reshape(K, NWORK, rpw // CHUNK, CHUNK).transpose(1, 0, 2, 3)
    sc = _make_sc_gather_sum(B * N, CO, n_workers=NWORK, chunk=CHUNK,
                             n_taps=K)
    out_rows = sc(table, idx_planes)
    return out_rows.reshape(B, N, CO).transpose(0, 2, 1)
